# Initial kernel scaffold; baseline (speedup 1.0000x reference)
import jax
import jax.numpy as jnp
from jax.experimental import pallas as pl


def kernel(inputs, targets, smooth):
    raise NotImplementedError("write your pallas kernel here")


# speedup vs baseline: 2.3596x; 2.3596x over previous
import jax
import jax.numpy as jnp
from jax.experimental import pallas as pl
from jax.experimental.pallas import tpu as pltpu


def _dice_body(smooth_ref, inp_ref, tgt_ref, out_ref, acc_ref):
    n = pl.program_id(0)
    num_n = pl.num_programs(0)
    x = inp_ref[0]
    t = tgt_ref[0]
    smooth = smooth_ref[0, 0]
    C = x.shape[0]

    @pl.when(n == 0)
    def _init():
        acc_ref[0] = 0.0

    r = jnp.float32(0.0)
    for c in range(C):
        xc = x[c]
        m = (t == c).astype(jnp.float32)
        inter = jnp.sum(m * xc)
        xsum = jnp.sum(xc)
        cnt = jnp.sum(m)
        r = r + (2.0 * inter + smooth) / (xsum + cnt + smooth)
    acc_ref[0] = acc_ref[0] + r

    @pl.when(n == num_n - 1)
    def _fini():
        out_ref[0, 0] = 1.0 - acc_ref[0] / (num_n * C)


def kernel(inputs, targets, smooth):
    N, C, H, W = inputs.shape
    t32 = targets.astype(jnp.int32)
    s = jnp.asarray(smooth, jnp.float32).reshape(1, 1)
    out = pl.pallas_call(
        _dice_body,
        grid=(N,),
        in_specs=[
            pl.BlockSpec(memory_space=pltpu.SMEM),
            pl.BlockSpec((1, C, H, W), lambda n: (n, 0, 0, 0)),
            pl.BlockSpec((1, H, W), lambda n: (n, 0, 0)),
        ],
        out_specs=pl.BlockSpec(memory_space=pltpu.SMEM),
        out_shape=jax.ShapeDtypeStruct((1, 1), jnp.float32),
        scratch_shapes=[pltpu.SMEM((1,), jnp.float32)],
    )(s, inputs, t32)
    return out[0, 0]
